# baseline (speedup 1.0000x reference)
import jax
import jax.numpy as jnp
from jax import lax
from jax.experimental import pallas as pl
from jax.experimental.pallas import tpu as pltpu

S, D, E, F = 2048, 1024, 8, 512


def _moe_dense_kernel(x_ref, wg_ref, w1_ref, w2_ref, w3_ref, out_ref, g_ref):
    j = pl.program_id(0)
    xs = x_ref[...]

    @pl.when(j == 0)
    def _():
        scores = jnp.dot(xs, wg_ref[...], preferred_element_type=jnp.float32)
        iota = lax.broadcasted_iota(jnp.int32, scores.shape, 1)
        m1 = jnp.max(scores, axis=-1, keepdims=True)
        idx1 = jnp.min(jnp.where(scores == m1, iota, E), axis=-1, keepdims=True)
        oh1 = iota == idx1
        scores2 = jnp.where(oh1, -jnp.inf, scores)
        m2 = jnp.max(scores2, axis=-1, keepdims=True)
        idx2 = jnp.min(jnp.where(scores2 == m2, iota, E), axis=-1, keepdims=True)
        oh2 = iota == idx2
        t = jnp.exp(m2 - m1)
        p1 = 1.0 / (1.0 + t)
        p2 = t / (1.0 + t)
        g_ref[...] = p1 * oh1.astype(jnp.float32) + p2 * oh2.astype(jnp.float32)

    ie = lax.broadcasted_iota(jnp.int32, (E, 2), 0)
    ic = lax.broadcasted_iota(jnp.int32, (E, 2), 1)
    sel = (ie == 2 * j + ic).astype(jnp.float32)
    gates2 = jnp.dot(g_ref[...], sel, preferred_element_type=jnp.float32)

    for half in range(4):
        rows = pl.ds(half * (S // 4), S // 4)
        xh = x_ref[rows, :]
        a0 = jnp.dot(xh, w1_ref[0], preferred_element_type=jnp.float32)
        b0 = jnp.dot(xh, w2_ref[0], preferred_element_type=jnp.float32)
        h0 = (a0 * lax.logistic(a0)) * b0 * gates2[half * (S // 4):(half + 1) * (S // 4), 0:1]
        y = jnp.dot(h0, w3_ref[0], preferred_element_type=jnp.float32)
        a1 = jnp.dot(xh, w1_ref[1], preferred_element_type=jnp.float32)
        b1 = jnp.dot(xh, w2_ref[1], preferred_element_type=jnp.float32)
        h1 = (a1 * lax.logistic(a1)) * b1 * gates2[half * (S // 4):(half + 1) * (S // 4), 1:2]
        y = y + jnp.dot(h1, w3_ref[1], preferred_element_type=jnp.float32)

        @pl.when(j == 0)
        def _():
            out_ref[rows, :] = y

        @pl.when(j > 0)
        def _():
            out_ref[rows, :] += y


def kernel(x, Wg, W1, W2, W3):
    B = x.shape[0]
    xs = x.reshape(S, D)

    out = pl.pallas_call(
        _moe_dense_kernel,
        grid=(E // 2,),
        in_specs=[
            pl.BlockSpec((S, D), lambda j: (0, 0)),
            pl.BlockSpec((D, E), lambda j: (0, 0)),
            pl.BlockSpec((2, D, F), lambda j: (j, 0, 0)),
            pl.BlockSpec((2, D, F), lambda j: (j, 0, 0)),
            pl.BlockSpec((2, F, D), lambda j: (j, 0, 0)),
        ],
        out_specs=pl.BlockSpec((S, D), lambda j: (0, 0)),
        out_shape=jax.ShapeDtypeStruct((S, D), jnp.float32),
        scratch_shapes=[pltpu.VMEM((S, E), jnp.float32)],
    )(xs, Wg, W1, W2, W3)
    return out.reshape(B, S, D)


# speedup vs baseline: 1.0643x; 1.0643x over previous
import jax
import jax.numpy as jnp
from jax import lax
from jax.experimental import pallas as pl
from jax.experimental.pallas import tpu as pltpu

S, D, E, F = 2048, 1024, 8, 512


def _moe_dense_kernel(x_ref, wg_ref, w1_ref, w2_ref, w3_ref, out_ref, g_ref):
    j = pl.program_id(0)
    xs = x_ref[...]

    @pl.when(j == 0)
    def _():
        scores = jnp.dot(xs, wg_ref[...], preferred_element_type=jnp.float32)
        iota = lax.broadcasted_iota(jnp.int32, scores.shape, 1)
        m1 = jnp.max(scores, axis=-1, keepdims=True)
        idx1 = jnp.min(jnp.where(scores == m1, iota, E), axis=-1, keepdims=True)
        oh1 = iota == idx1
        scores2 = jnp.where(oh1, -jnp.inf, scores)
        m2 = jnp.max(scores2, axis=-1, keepdims=True)
        idx2 = jnp.min(jnp.where(scores2 == m2, iota, E), axis=-1, keepdims=True)
        oh2 = iota == idx2
        t = jnp.exp(m2 - m1)
        p1 = 1.0 / (1.0 + t)
        p2 = t / (1.0 + t)
        g_ref[...] = p1 * oh1.astype(jnp.float32) + p2 * oh2.astype(jnp.float32)

    iota = lax.broadcasted_iota(jnp.int32, (S, E), 1)
    g = g_ref[...]
    ga = jnp.sum(jnp.where(iota == 2 * j, g, 0.0), axis=-1, keepdims=True)
    gb = jnp.sum(jnp.where(iota == 2 * j + 1, g, 0.0), axis=-1, keepdims=True)

    for half in range(2):
        rows = pl.ds(half * (S // 2), S // 2)
        xh = x_ref[rows, :]
        a0 = jnp.dot(xh, w1_ref[0], preferred_element_type=jnp.float32)
        b0 = jnp.dot(xh, w2_ref[0], preferred_element_type=jnp.float32)
        h0 = (a0 * lax.logistic(a0)) * b0 * ga[half * (S // 2):(half + 1) * (S // 2)]
        y = jnp.dot(h0, w3_ref[0], preferred_element_type=jnp.float32)
        a1 = jnp.dot(xh, w1_ref[1], preferred_element_type=jnp.float32)
        b1 = jnp.dot(xh, w2_ref[1], preferred_element_type=jnp.float32)
        h1 = (a1 * lax.logistic(a1)) * b1 * gb[half * (S // 2):(half + 1) * (S // 2)]
        y = y + jnp.dot(h1, w3_ref[1], preferred_element_type=jnp.float32)

        @pl.when(j == 0)
        def _():
            out_ref[rows, :] = y

        @pl.when(j > 0)
        def _():
            out_ref[rows, :] += y


def kernel(x, Wg, W1, W2, W3):
    B = x.shape[0]
    xs = x.reshape(S, D)

    out = pl.pallas_call(
        _moe_dense_kernel,
        grid=(E // 2,),
        in_specs=[
            pl.BlockSpec((S, D), lambda j: (0, 0)),
            pl.BlockSpec((D, E), lambda j: (0, 0)),
            pl.BlockSpec((2, D, F), lambda j: (j, 0, 0)),
            pl.BlockSpec((2, D, F), lambda j: (j, 0, 0)),
            pl.BlockSpec((2, F, D), lambda j: (j, 0, 0)),
        ],
        out_specs=pl.BlockSpec((S, D), lambda j: (0, 0)),
        out_shape=jax.ShapeDtypeStruct((S, D), jnp.float32),
        scratch_shapes=[pltpu.VMEM((S, E), jnp.float32)],
    )(xs, Wg, W1, W2, W3)
    return out.reshape(B, S, D)
